# Initial kernel scaffold; baseline (speedup 1.0000x reference)
import jax
import jax.numpy as jnp
from jax.experimental import pallas as pl


def kernel(seq, mask, adj_matrix, W, W_self, b):
    raise NotImplementedError("write your pallas kernel here")


# speedup vs baseline: 1408.8656x; 1408.8656x over previous
import jax
import jax.numpy as jnp
from jax.experimental import pallas as pl


def _mp_kernel(seq_ref, adj_ref, w_ref, ws_ref, b_ref, out_ref):
    x = seq_ref[0]
    a = (adj_ref[0] != 0).astype(jnp.float32)
    h = jnp.dot(x, w_ref[...], preferred_element_type=jnp.float32)
    agg = jax.lax.dot_general(
        a, h, (((0,), (0,)), ((), ())), preferred_element_type=jnp.float32
    )
    self_term = jnp.dot(x, ws_ref[...], preferred_element_type=jnp.float32)
    out_ref[0] = agg + self_term + b_ref[...]


def kernel(seq, mask, adj_matrix, W, W_self, b):
    B, L, d = seq.shape
    del mask
    b2d = b.reshape(1, d)
    out = pl.pallas_call(
        _mp_kernel,
        grid=(B,),
        in_specs=[
            pl.BlockSpec((1, L, d), lambda i: (i, 0, 0)),
            pl.BlockSpec((1, L, L), lambda i: (i, 0, 0)),
            pl.BlockSpec((d, d), lambda i: (0, 0)),
            pl.BlockSpec((d, d), lambda i: (0, 0)),
            pl.BlockSpec((1, d), lambda i: (0, 0)),
        ],
        out_specs=pl.BlockSpec((1, L, d), lambda i: (i, 0, 0)),
        out_shape=jax.ShapeDtypeStruct((B, L, d), jnp.float32),
    )(seq, adj_matrix, W, W_self, b2d)
    return out
